# Initial kernel scaffold; baseline (speedup 1.0000x reference)
import jax
import jax.numpy as jnp
from jax import lax
from jax.experimental import pallas as pl
from jax.experimental.pallas import tpu as pltpu
from jax.experimental.pallas import tpu_sc as plsc

N = 100000
D = 128
NUM_Z = 119

_INFO = plsc.get_sparse_core_info()
_NC = _INFO.num_cores
_NS = _INFO.num_subcores
_NW = _NC * _NS

R = 80
NCHUNKS = N // R


def _sc_body(inp_hbm, z_hbm, scale_hbm, shift_hbm, out_hbm,
             buf, zbuf, scale_v, shift_v):
    wid = lax.axis_index("s") * _NC + lax.axis_index("c")

    pltpu.sync_copy(scale_hbm, scale_v)
    pltpu.sync_copy(shift_hbm, shift_v)

    nmine = (NCHUNKS + (_NW - 1) - wid) // _NW

    def chunk_body(i, carry):
        cid = wid + i * _NW
        base = cid * R
        pltpu.sync_copy(inp_hbm.at[pl.ds(base, R)], buf)
        pltpu.sync_copy(z_hbm.at[pl.ds(base, R)], zbuf)

        def row_body(r, c2):
            zs = zbuf[r]
            s = scale_v[zs]
            h = shift_v[zs]
            for j in range(D // 16):
                v = buf[r, pl.ds(j * 16, 16)]
                buf[r, pl.ds(j * 16, 16)] = v * s + h
            return c2

        lax.fori_loop(0, R, row_body, 0)
        pltpu.sync_copy(buf, out_hbm.at[pl.ds(base, R)])
        return carry

    lax.fori_loop(0, nmine, chunk_body, 0)


@jax.jit
def _scale_shift(inp, z, scale_flat, shift_flat):
    mesh = plsc.VectorSubcoreMesh(core_axis_name="c", subcore_axis_name="s")
    f = pl.kernel(
        _sc_body,
        out_type=jax.ShapeDtypeStruct((N, D), jnp.float32),
        mesh=mesh,
        scratch_types=[
            pltpu.VMEM((R, D), jnp.float32),
            pltpu.VMEM((R,), jnp.int32),
            pltpu.VMEM((NUM_Z,), jnp.float32),
            pltpu.VMEM((NUM_Z,), jnp.float32),
        ],
    )
    return f(inp, z, scale_flat, shift_flat)


def kernel(input, z, scale_table, shift_table):
    scale_flat = scale_table.reshape(NUM_Z)
    shift_flat = shift_table.reshape(NUM_Z)
    return _scale_shift(input, z, scale_flat, shift_flat)


# speedup vs baseline: 19.7650x; 19.7650x over previous
import jax
import jax.numpy as jnp
from jax import lax
from jax.experimental import pallas as pl
from jax.experimental.pallas import tpu as pltpu
from jax.experimental.pallas import tpu_sc as plsc

N = 100000
D = 128
NUM_Z = 119

_INFO = plsc.get_sparse_core_info()
_NC = _INFO.num_cores
_NS = _INFO.num_subcores
_NW = _NC * _NS

R = 160
NCHUNKS = N // R
JMAX = (NCHUNKS + _NW - 1) // _NW


def _sc_body(inp_hbm, z_hbm, scale_hbm, shift_hbm, out_hbm,
             bufin, bufout, zin0, zin1, scale_v, shift_v,
             insem0, insem1, outsem0, outsem1):
    wid = lax.axis_index("s") * _NC + lax.axis_index("c")
    insems = (insem0, insem1)
    outsems = (outsem0, outsem1)
    zins = (zin0, zin1)

    pltpu.sync_copy(scale_hbm, scale_v)
    pltpu.sync_copy(shift_hbm, shift_v)

    def valid(j):
        cid = wid + j * _NW
        return jnp.logical_and(j >= 0, cid < NCHUNKS)

    def fill(j, slot):
        cid = wid + j * _NW

        @pl.when(valid(j))
        def _():
            base = cid * R
            pltpu.async_copy(inp_hbm.at[pl.ds(base, R)], bufin.at[slot],
                             insems[slot])
            pltpu.async_copy(z_hbm.at[pl.ds(base, R)], zins[slot],
                             insems[slot])

    def wait_fill(j, slot):
        @pl.when(valid(j))
        def _():
            pltpu.make_async_copy(inp_hbm.at[pl.ds(0, R)], bufin.at[slot],
                                  insems[slot]).wait()
            pltpu.make_async_copy(z_hbm.at[pl.ds(0, R)], zins[slot],
                                  insems[slot]).wait()

    def compute(j, slot):
        @pl.when(valid(j))
        def _():
            def group_body(g, c2):
                zvec = zins[slot][pl.ds(g * 16, 16)]
                svec = plsc.load_gather(scale_v, [zvec])
                hvec = plsc.load_gather(shift_v, [zvec])
                row0 = g * 16
                for rr in range(16):
                    s = svec[rr]
                    h = hvec[rr]
                    r = row0 + rr
                    for jj in range(D // 16):
                        v = bufin[slot, r, pl.ds(jj * 16, 16)]
                        bufout[slot, r, pl.ds(jj * 16, 16)] = v * s + h
                return c2

            lax.fori_loop(0, R // 16, group_body, 0)

    def drain(j, slot):
        cid = wid + j * _NW

        @pl.when(valid(j))
        def _():
            base = cid * R
            pltpu.async_copy(bufout.at[slot], out_hbm.at[pl.ds(base, R)],
                             outsems[slot])

    def wait_drain(j, slot):
        @pl.when(valid(j))
        def _():
            pltpu.make_async_copy(bufout.at[slot], out_hbm.at[pl.ds(0, R)],
                                  outsems[slot]).wait()

    fill(jnp.int32(0), 0)

    def body(i, carry):
        j0 = 2 * i
        j1 = 2 * i + 1
        fill(j1, 1)
        wait_fill(j0, 0)
        wait_drain(j0 - 2, 0)
        compute(j0, 0)
        drain(j0, 0)
        fill(j0 + 2, 0)
        wait_fill(j1, 1)
        wait_drain(j1 - 2, 1)
        compute(j1, 1)
        drain(j1, 1)
        return carry

    lax.fori_loop(0, JMAX // 2, body, 0)
    wait_drain(jnp.int32(JMAX - 2), 0)
    wait_drain(jnp.int32(JMAX - 1), 1)


@jax.jit
def _scale_shift(inp, z, scale_flat, shift_flat):
    mesh = plsc.VectorSubcoreMesh(core_axis_name="c", subcore_axis_name="s")
    f = pl.kernel(
        _sc_body,
        out_type=jax.ShapeDtypeStruct((N, D), jnp.float32),
        mesh=mesh,
        compiler_params=pltpu.CompilerParams(needs_layout_passes=False),
        scratch_types=[
            pltpu.VMEM((2, R, D), jnp.float32),
            pltpu.VMEM((2, R, D), jnp.float32),
            pltpu.VMEM((R,), jnp.int32),
            pltpu.VMEM((R,), jnp.int32),
            pltpu.VMEM((128,), jnp.float32),
            pltpu.VMEM((128,), jnp.float32),
            pltpu.SemaphoreType.DMA,
            pltpu.SemaphoreType.DMA,
            pltpu.SemaphoreType.DMA,
            pltpu.SemaphoreType.DMA,
        ],
    )
    return f(inp, z, scale_flat, shift_flat)


def kernel(input, z, scale_table, shift_table):
    scale_flat = jnp.zeros((128,), jnp.float32).at[:NUM_Z].set(scale_table.reshape(NUM_Z))
    shift_flat = jnp.zeros((128,), jnp.float32).at[:NUM_Z].set(shift_table.reshape(NUM_Z))
    return _scale_shift(input, z, scale_flat, shift_flat)
